# baseline (speedup 1.0000x reference)
import jax
import jax.numpy as jnp
from jax import lax
from jax.experimental import pallas as pl
from jax.experimental.pallas import tpu as pltpu
from jax.experimental.pallas import tpu_sc as plsc

N_NODES = 10000
D = 128
NC = 2
NS = 16
HC = D // NC
R = 640
RP = R // 2
NPAIR = N_NODES // 2
PPAD = NS * RP
C = 6400
L = 16
U = 8
W = 64
SHIFT = 14


def _body(ep_hbm, x_hbm, out_hbm,
          acc, ec, sel_s, sel_d, rows0, rows1, xs, sem0, sem1):
    E = ep_hbm.shape[0]
    n_chunks = E // C
    cid = lax.axis_index("c")
    sid = lax.axis_index("s")
    wid = sid * NC + cid
    lo = sid * R
    minus_inf = jnp.full((L,), -jnp.inf, jnp.float32)

    xbase = cid * NPAIR
    @pl.when(sid < NS - 1)
    def _():
        pltpu.sync_copy(x_hbm.at[pl.ds(xbase + sid * 320, 320)],
                        xs.at[pl.ds(sid * 320, 320)])
    @pl.when(sid == NS - 1)
    def _():
        rem = NPAIR - (NS - 1) * 320
        pltpu.sync_copy(
            x_hbm.at[pl.ds(xbase + (NS - 1) * 320, rem)],
            xs.at[pl.ds((NS - 1) * 320, rem)])

    def init_row(r, _):
        for k in range(D // L):
            acc[r, pl.ds(k * L, L)] = minus_inf
        return 0
    lax.fori_loop(0, RP + 1, init_row, 0)
    plsc.subcore_barrier()

    lob = lo << SHIFT
    bufs = ((rows0, sem0), (rows1, sem1))

    def fire(w, buf, sem):
        pltpu.async_copy(xs.at[sel_s.at[pl.ds(w * W, W)]], buf, sem)

    def do_chunk(ci, _):
        base = ci * C
        pltpu.sync_copy(ep_hbm.at[pl.ds(base, C)], ec)

        rspan = jnp.uint32(R << SHIFT)

        def scanU(i, cnt_vec):
            for u in range(U):
                p = ec[pl.ds((i * U + u) * L, L)]
                q = p - lob
                m = q.astype(jnp.uint32) < rspan
                mi = m.astype(jnp.int32)
                incl = plsc.cumsum(mi)
                pos = cnt_vec + (incl - mi)
                s = p & ((1 << SHIFT) - 1)
                v2 = (q >> SHIFT) | ((s & 1) << 12)
                plsc.store_scatter(sel_s, [pos], s >> 1, mask=m)
                plsc.store_scatter(sel_d, [pos], v2, mask=m)
                cnt_vec = cnt_vec + plsc.all_reduce_population_count(m)
            return cnt_vec
        cnt_vec = lax.fori_loop(0, C // (L * U), scanU,
                                jnp.zeros((L,), jnp.int32))
        n = cnt_vec[0]

        padv = jnp.full((L,), wid, jnp.int32)
        padd = jnp.full((L,), R, jnp.int32)
        for j in range(W // L):
            sel_s[pl.ds(n + j * L, L)] = padv
            sel_d[pl.ds(n + j * L, L)] = padd

        ng = (n + L - 1) // L
        nw = (ng + W // L - 1) // (W // L)

        @pl.when(nw > 0)
        def _():
            fire(0, rows0, sem0)
        @pl.when(nw > 1)
        def _():
            fire(1, rows1, sem1)

        def pair(wp, _):
            for b in range(2):
                rows, sem = bufs[b]
                w = wp * 2 + b

                @pl.when(w < nw)
                def _():
                    pltpu.make_async_copy(
                        xs.at[sel_s.at[pl.ds(w * W, W)]], rows, sem).wait()
                    gend = jnp.minimum(W // L, ng - (W // L) * w)

                    def grp(j, _):
                        goff = w * W + j * L
                        dl = sel_d[pl.ds(goff, L)]
                        for lane in range(L):
                            v2 = dl[lane]
                            dloc = v2 & 0xFFF
                            dr = dloc >> 1
                            cbd = (dloc & 1) << 6
                            cbs = ((v2 >> 12) & 1) << 6
                            rr = j * L + lane
                            for k in range(HC // L):
                                sd = pl.ds(cbd + k * L, L)
                                ss = pl.ds(cbs + k * L, L)
                                acc[dr, sd] = jnp.maximum(acc[dr, sd],
                                                          rows[rr, ss])
                        return 0
                    lax.fori_loop(0, gend, grp, 0)

                    @pl.when(w + 2 < nw)
                    def _():
                        fire(w + 2, rows, sem)
            return 0
        lax.fori_loop(0, (nw + 1) // 2, pair, 0)
        return 0
    lax.fori_loop(0, n_chunks, do_chunk, 0)

    zeros = jnp.zeros((L,), jnp.float32)
    def fix_row(r, _):
        for k in range(D // L):
            sl = pl.ds(k * L, L)
            v = acc[r, sl]
            acc[r, sl] = jnp.where(v == -jnp.inf, zeros, v)
        return 0
    lax.fori_loop(0, RP, fix_row, 0)
    pltpu.sync_copy(acc.at[pl.ds(0, RP)],
                    out_hbm.at[pl.ds(cid * PPAD + sid * RP, RP)])


def kernel(x, edge_index):
    edge_index = edge_index.astype(jnp.int32)
    ep = edge_index[0] | (edge_index[1] << SHIFT)
    x2 = (x.reshape(NPAIR, 2, NC, HC).transpose(2, 0, 1, 3)
          .reshape(NC * NPAIR, D))
    mesh = plsc.VectorSubcoreMesh(
        core_axis_name="c", subcore_axis_name="s",
        num_cores=NC, num_subcores=NS)
    f = pl.kernel(
        _body,
        out_type=jax.ShapeDtypeStruct((NC * PPAD, D), jnp.float32),
        mesh=mesh,
        compiler_params=pltpu.CompilerParams(needs_layout_passes=False),
        scratch_types=[
            pltpu.VMEM((RP + 1, D), jnp.float32),
            pltpu.VMEM((C,), jnp.int32),
            pltpu.VMEM((C + W,), jnp.int32),
            pltpu.VMEM((C + W,), jnp.int32),
            pltpu.VMEM((W, D), jnp.float32),
            pltpu.VMEM((W, D), jnp.float32),
            pltpu.VMEM_SHARED((NPAIR, D), jnp.float32),
            pltpu.SemaphoreType.DMA,
            pltpu.SemaphoreType.DMA,
        ],
    )
    out = f(ep, x2)
    out = (out.reshape(NC, PPAD, 2, HC).transpose(1, 2, 0, 3)
           .reshape(2 * PPAD, D))
    return out[:N_NODES]


# speedup vs baseline: 1.1618x; 1.1618x over previous
import jax
import jax.numpy as jnp
from jax import lax
from jax.experimental import pallas as pl
from jax.experimental.pallas import tpu as pltpu
from jax.experimental.pallas import tpu_sc as plsc

N_NODES = 10000
D = 128
NC = 2
NS = 16
NW = NC * NS
R = 320
N_PAD = NW * R
C = 12800
L = 16
U = 8
W = 64
NB = 4
SHIFT = 14


def _body(ep_hbm, x_hbm, out_hbm,
          acc, ec, sel_s, sel_d, rows0, rows1, rows2, rows3,
          sem0, sem1, sem2, sem3):
    E = ep_hbm.shape[0]
    n_chunks = E // C
    cid = lax.axis_index("c")
    sid = lax.axis_index("s")
    wid = sid * NC + cid
    lo = wid * R
    minus_inf = jnp.full((L,), -jnp.inf, jnp.float32)

    def init_row(r, _):
        for k in range(D // L):
            acc[r, pl.ds(k * L, L)] = minus_inf
        return 0
    lax.fori_loop(0, R + 1, init_row, 0)

    lob = lo << SHIFT
    hib = (lo + R) << SHIFT
    bufs = ((rows0, sem0), (rows1, sem1), (rows2, sem2), (rows3, sem3))

    def fire(w, buf, sem):
        pltpu.async_copy(x_hbm.at[sel_s.at[pl.ds(w * W, W)]], buf, sem)

    def do_chunk(ci, _):
        base = ci * C
        pltpu.sync_copy(ep_hbm.at[pl.ds(base, C)], ec)

        rspan = jnp.uint32(R << SHIFT)

        def scanU(i, cnt_vec):
            for u in range(U):
                p = ec[pl.ds((i * U + u) * L, L)]
                q = p - lob
                m = q.astype(jnp.uint32) < rspan
                mi = m.astype(jnp.int32)
                incl = plsc.cumsum(mi)
                pos = cnt_vec + (incl - mi)
                plsc.store_scatter(sel_s, [pos], p & ((1 << SHIFT) - 1),
                                   mask=m)
                plsc.store_scatter(sel_d, [pos], q >> SHIFT, mask=m)
                cnt_vec = cnt_vec + plsc.all_reduce_population_count(m)
            return cnt_vec
        cnt_vec = lax.fori_loop(0, C // (L * U), scanU,
                                jnp.zeros((L,), jnp.int32))
        n = cnt_vec[0]

        padv = jnp.full((L,), wid, jnp.int32)
        padd = jnp.full((L,), R, jnp.int32)
        for j in range(W // L):
            sel_s[pl.ds(n + j * L, L)] = padv
            sel_d[pl.ds(n + j * L, L)] = padd

        ng = (n + L - 1) // L
        nw = (ng + W // L - 1) // (W // L)

        for b in range(NB):
            @pl.when(nw > b)
            def _(b=b):
                fire(b, bufs[b][0], bufs[b][1])

        def quad(wp, _):
            for b in range(NB):
                rows, sem = bufs[b]
                w = wp * NB + b

                @pl.when(w < nw)
                def _():
                    pltpu.make_async_copy(
                        x_hbm.at[sel_s.at[pl.ds(w * W, W)]], rows, sem).wait()
                    gend = jnp.minimum(W // L, ng - (W // L) * w)

                    def grp(j, _):
                        goff = w * W + j * L
                        dl = sel_d[pl.ds(goff, L)]
                        for lane in range(L):
                            dr = dl[lane]
                            rr = j * L + lane
                            for k in range(D // L):
                                sl = pl.ds(k * L, L)
                                acc[dr, sl] = jnp.maximum(acc[dr, sl],
                                                          rows[rr, sl])
                        return 0
                    lax.fori_loop(0, gend, grp, 0)

                    @pl.when(w + NB < nw)
                    def _():
                        fire(w + NB, rows, sem)
            return 0
        lax.fori_loop(0, (nw + NB - 1) // NB, quad, 0)
        return 0
    lax.fori_loop(0, n_chunks, do_chunk, 0)

    zeros = jnp.zeros((L,), jnp.float32)
    def fix_row(r, _):
        for k in range(D // L):
            sl = pl.ds(k * L, L)
            v = acc[r, sl]
            acc[r, sl] = jnp.where(v == -jnp.inf, zeros, v)
        return 0
    lax.fori_loop(0, R, fix_row, 0)
    pltpu.sync_copy(acc.at[pl.ds(0, R)], out_hbm.at[pl.ds(lo, R)])


def kernel(x, edge_index):
    edge_index = edge_index.astype(jnp.int32)
    ep = edge_index[0] | (edge_index[1] << SHIFT)
    mesh = plsc.VectorSubcoreMesh(
        core_axis_name="c", subcore_axis_name="s",
        num_cores=NC, num_subcores=NS)
    f = pl.kernel(
        _body,
        out_type=jax.ShapeDtypeStruct((N_PAD, D), jnp.float32),
        mesh=mesh,
        compiler_params=pltpu.CompilerParams(needs_layout_passes=False),
        scratch_types=[
            pltpu.VMEM((R + 1, D), jnp.float32),
            pltpu.VMEM((C,), jnp.int32),
            pltpu.VMEM((C + W,), jnp.int32),
            pltpu.VMEM((C + W,), jnp.int32),
            pltpu.VMEM((W, D), jnp.float32),
            pltpu.VMEM((W, D), jnp.float32),
            pltpu.VMEM((W, D), jnp.float32),
            pltpu.VMEM((W, D), jnp.float32),
            pltpu.SemaphoreType.DMA,
            pltpu.SemaphoreType.DMA,
            pltpu.SemaphoreType.DMA,
            pltpu.SemaphoreType.DMA,
        ],
    )
    out = f(ep, x)
    return out[:N_NODES]
